# baseline (speedup 1.0000x reference)
import functools

import jax
import jax.numpy as jnp
from jax.experimental import pallas as pl
from jax.experimental.pallas import tpu as pltpu

N = 10000
E = 160000


def _linear_kernel(a_ref, w_ref, b_ref, o_ref, *, relu, normalize):
    acc = jnp.dot(a_ref[...], w_ref[...], preferred_element_type=jnp.float32)
    acc = acc + b_ref[...]
    if relu:
        acc = jnp.maximum(acc, 0.0)
    if normalize:
        norm = jnp.sqrt(jnp.sum(acc * acc, axis=1, keepdims=True))
        acc = acc / jnp.maximum(norm, 1e-12)
    o_ref[...] = acc


def _linear(a, w, b, *, relu=True, normalize=False, bm=512, bn=512):
    M, K = a.shape
    C = w.shape[0]
    if normalize:
        bn = C
    wT = w.T
    b2 = b.reshape(1, C)
    Mp = (M + bm - 1) // bm * bm
    if Mp != M:
        a = jnp.pad(a, ((0, Mp - M), (0, 0)))
    grid = (Mp // bm, C // bn)
    out = pl.pallas_call(
        functools.partial(_linear_kernel, relu=relu, normalize=normalize),
        grid=grid,
        in_specs=[
            pl.BlockSpec((bm, K), lambda i, j: (i, 0)),
            pl.BlockSpec((K, bn), lambda i, j: (0, j)),
            pl.BlockSpec((1, bn), lambda i, j: (0, j)),
        ],
        out_specs=pl.BlockSpec((bm, bn), lambda i, j: (i, j)),
        out_shape=jax.ShapeDtypeStruct((Mp, C), jnp.float32),
    )(a, wT, b2)
    return out[:M]


def _edge_stage(h, rel, src, dst, w_in, b_in):
    scaling = jax.nn.relu(rel @ w_in.T + b_in)
    msg = scaling * h[src]
    return jax.ops.segment_sum(msg, dst, num_segments=N)


def kernel(x, pos, edge_index, w_in1, b_in1, w_out1, b_out1,
           w_in2, b_in2, w_out2, b_out2, w_in3, b_in3, w_out3, b_out3):
    src, dst = edge_index[0], edge_index[1]
    rel = pos[src] - pos[dst]

    a1 = _edge_stage(x, rel, src, dst, w_in1, b_in1)
    h1 = _linear(a1, w_out1, b_out1, relu=True)
    a2 = _edge_stage(h1, rel, src, dst, w_in2, b_in2)
    h2 = _linear(a2, w_out2, b_out2, relu=True)
    a3 = _edge_stage(h2, rel, src, dst, w_in3, b_in3)
    h3 = _linear(a3, w_out3, b_out3, relu=True, normalize=True)
    return h3
